# Initial kernel scaffold; baseline (speedup 1.0000x reference)
import jax
import jax.numpy as jnp
from jax.experimental import pallas as pl


def kernel(headers, payloads, header_table, header_ln_g, header_ln_b, conv_w, conv_ln_g, conv_ln_b, packet_table):
    raise NotImplementedError("write your pallas kernel here")


# speedup vs baseline: 4.5496x; 4.5496x over previous
import numpy as np
import jax
import jax.numpy as jnp
from jax.experimental import pallas as pl
from jax.experimental.pallas import tpu as pltpu

D = 768
EPS = 1e-5


def _make_pe(n):
    position = np.arange(n, dtype=np.float32)[:, None]
    div_term = np.exp(
        np.arange(0, D, 2, dtype=np.float32) * -(np.log(10000.0) / D)
    )
    pe = np.zeros((n, D), dtype=np.float32)
    pe[:, 0::2] = np.sin(position * div_term)
    pe[:, 1::2] = np.cos(position * div_term)
    return jnp.asarray(pe)


def _ln(y, g, b):
    m = jnp.mean(y, axis=-1, keepdims=True)
    yc = y - m
    v = jnp.mean(yc * yc, axis=-1, keepdims=True)
    return yc * jax.lax.rsqrt(v + EPS) * g + b


def _fused_body(
    idx_ref, x_ref, w_ref, tab_ref,
    hg_ref, hb_ref, cg_ref, cb_ref,
    pe_h_ref, pe_p_ref, pk_ref,
    h_out_ref, p_out_ref,
):
    pk = jnp.tanh(pk_ref[0])

    idx = idx_ref[0, 0]
    onehot = (idx == jax.lax.broadcasted_iota(jnp.int32, (idx.shape[0], 256), 1)
              ).astype(jnp.float32)
    h = jnp.dot(onehot, tab_ref[...], preferred_element_type=jnp.float32)
    h = _ln(h, hg_ref[...], hb_ref[...])
    h_out_ref[0, 0] = h + pe_h_ref[...] + pk

    x = x_ref[0, 0]
    xm = jnp.roll(x, 1, axis=0)
    xp = jnp.roll(x, -1, axis=0)
    w = w_ref[...]
    y = xm * w[0:1, :] + x * w[1:2, :] + xp * w[2:3, :]
    y = _ln(y, cg_ref[...], cb_ref[...])
    y = 0.5 * y * (1.0 + jax.lax.erf(y * np.float32(1.0 / np.sqrt(2.0))))
    p_out_ref[0, 0] = y + pe_p_ref[...] + pk


def kernel(headers, payloads, header_table, header_ln_g, header_ln_b,
           conv_w, conv_ln_g, conv_ln_b, packet_table):
    B, P, T = headers.shape
    L = payloads.shape[2]

    idx = headers.astype(jnp.int32).reshape(B, P, T, 1)
    x = payloads.reshape(B, P, L, 1)
    wmat = conv_w[:, 0, :].T
    pe_h = _make_pe(T)
    pe_p = _make_pe(L)

    const = lambda shape: pl.BlockSpec(shape, lambda b, p: (0,) * len(shape))

    h_out, p_out = pl.pallas_call(
        _fused_body,
        grid=(B, P),
        in_specs=[
            pl.BlockSpec((1, 1, T, 1), lambda b, p: (b, p, 0, 0)),
            pl.BlockSpec((1, 1, L, 1), lambda b, p: (b, p, 0, 0)),
            const((3, D)),
            const((256, D)),
            const((1, D)),
            const((1, D)),
            const((1, D)),
            const((1, D)),
            const((T, D)),
            const((L, D)),
            pl.BlockSpec((1, 1, D), lambda b, p: (p, 0, 0)),
        ],
        out_specs=[
            pl.BlockSpec((1, 1, T, D), lambda b, p: (b, p, 0, 0)),
            pl.BlockSpec((1, 1, L, D), lambda b, p: (b, p, 0, 0)),
        ],
        out_shape=[
            jax.ShapeDtypeStruct((B, P, T, D), jnp.float32),
            jax.ShapeDtypeStruct((B, P, L, D), jnp.float32),
        ],
    )(
        idx, x, wmat, header_table,
        header_ln_g.reshape(1, D), header_ln_b.reshape(1, D),
        conv_ln_g.reshape(1, D), conv_ln_b.reshape(1, D),
        pe_h, pe_p, packet_table.reshape(P, 1, D),
    )
    return h_out, p_out


# speedup vs baseline: 6.2309x; 1.3695x over previous
import numpy as np
import jax
import jax.numpy as jnp
from jax.experimental import pallas as pl
from jax.experimental.pallas import tpu as pltpu

D = 768
EPS = 1e-5


def _make_pe(n):
    position = np.arange(n, dtype=np.float32)[:, None]
    div_term = np.exp(
        np.arange(0, D, 2, dtype=np.float32) * -(np.log(10000.0) / D)
    )
    pe = np.zeros((n, D), dtype=np.float32)
    pe[:, 0::2] = np.sin(position * div_term)
    pe[:, 1::2] = np.cos(position * div_term)
    return jnp.asarray(pe)


def _ln(y, g, b):
    m = jnp.mean(y, axis=-1, keepdims=True)
    yc = y - m
    v = jnp.mean(yc * yc, axis=-1, keepdims=True)
    return yc * jax.lax.rsqrt(v + EPS) * g + b


def _fused_body(
    idx_ref, x_ref, w_ref, tab_ref,
    hg_ref, hb_ref, cg_ref, cb_ref,
    pe_h_ref, pe_p_ref, pk_ref,
    h_out_ref, p_out_ref,
):
    pk = jnp.tanh(pk_ref[0])

    idx = idx_ref[0, 0]
    onehot = (idx == jax.lax.broadcasted_iota(jnp.int32, (idx.shape[0], 256), 1)
              ).astype(jnp.float32)
    h = jnp.dot(onehot, tab_ref[...], preferred_element_type=jnp.float32)
    h = _ln(h, hg_ref[...], hb_ref[...])
    h_out_ref[0, 0] = h + pe_h_ref[...] + pk

    x = x_ref[0, 0]
    xm = jnp.roll(x, 1, axis=0)
    xp = jnp.roll(x, -1, axis=0)
    x3 = jnp.concatenate([xm, x, xp], axis=1)
    w = w_ref[...]
    s = jnp.sum(w, axis=1, keepdims=True) * np.float32(1.0 / D)
    gram = jax.lax.dot_general(
        w, w, (((1,), (1,)), ((), ())), preferred_element_type=jnp.float32
    ) * np.float32(1.0 / D)
    outer_s = jax.lax.dot_general(
        s, s, (((1,), (1,)), ((), ())), preferred_element_type=jnp.float32
    )
    g4 = jnp.concatenate([gram - outer_s, s], axis=1)
    t4 = jnp.dot(x3, g4, preferred_element_type=jnp.float32)
    v = jnp.sum(x3 * t4[:, :3], axis=1, keepdims=True)
    m = t4[:, 3:4]
    r = jax.lax.rsqrt(v + EPS)
    x5 = jnp.concatenate([x3 * r, -(m * r), jnp.ones_like(x)], axis=1)
    w5 = jnp.concatenate(
        [w * cg_ref[...], cg_ref[...], cb_ref[...]], axis=0
    )
    z = jnp.dot(x5, w5, preferred_element_type=jnp.float32)
    e = jax.lax.erf(z * np.float32(1.0 / np.sqrt(2.0)))
    zz = z * (0.5 * e + 0.5)
    p_out_ref[0, 0] = zz + pe_p_ref[...] + pk


def kernel(headers, payloads, header_table, header_ln_g, header_ln_b,
           conv_w, conv_ln_g, conv_ln_b, packet_table):
    B, P, T = headers.shape
    L = payloads.shape[2]

    idx = headers.astype(jnp.int32).reshape(B, P, T, 1)
    x = payloads.reshape(B, P, L, 1)
    wmat = conv_w[:, 0, :].T
    pe_h = _make_pe(T)
    pe_p = _make_pe(L)

    const = lambda shape: pl.BlockSpec(shape, lambda b, p: (0,) * len(shape))

    h_out, p_out = pl.pallas_call(
        _fused_body,
        grid=(B, P),
        in_specs=[
            pl.BlockSpec((1, 1, T, 1), lambda b, p: (b, p, 0, 0)),
            pl.BlockSpec((1, 1, L, 1), lambda b, p: (b, p, 0, 0)),
            const((3, D)),
            const((256, D)),
            const((1, D)),
            const((1, D)),
            const((1, D)),
            const((1, D)),
            const((T, D)),
            const((L, D)),
            pl.BlockSpec((1, 1, D), lambda b, p: (p, 0, 0)),
        ],
        out_specs=[
            pl.BlockSpec((1, 1, T, D), lambda b, p: (b, p, 0, 0)),
            pl.BlockSpec((1, 1, L, D), lambda b, p: (b, p, 0, 0)),
        ],
        out_shape=[
            jax.ShapeDtypeStruct((B, P, T, D), jnp.float32),
            jax.ShapeDtypeStruct((B, P, L, D), jnp.float32),
        ],
    )(
        idx, x, wmat, header_table,
        header_ln_g.reshape(1, D), header_ln_b.reshape(1, D),
        conv_ln_g.reshape(1, D), conv_ln_b.reshape(1, D),
        pe_h, pe_p, packet_table.reshape(P, 1, D),
    )
    return h_out, p_out
